# initial kernel scaffold (unmeasured)
import jax
import jax.numpy as jnp
from jax import lax
from jax.experimental import pallas as pl
from jax.experimental.pallas import tpu as pltpu

N_DEV = 16
B = 2
SQ = 256
SKV = 256
HQ = 4
DH = 64
CHUNK = HQ * DH
DM = 512
BS = B * SQ


def kernel(x, Wq, K_ext, V_ext, Wo):
    xb = x.astype(jnp.bfloat16)
    wqb = Wq.astype(jnp.bfloat16)
    wob = Wo.astype(jnp.bfloat16)
    k5 = K_ext.astype(jnp.bfloat16).reshape(B, SKV, N_DEV, HQ, DH)
    k5 = k5.transpose(2, 0, 1, 3, 4)
    v5 = V_ext.astype(jnp.bfloat16).reshape(B, SKV, N_DEV, HQ, DH)
    v5 = v5.transpose(2, 0, 1, 3, 4)

    def body(x_ref, wq_ref, k_ref, v_ref, wo_ref, out_ref,
             wq_bufs, wo_bufs, acc_ref, send_sems, recv_sems, credit_sem):
        my = lax.axis_index("i")
        left = lax.rem(my + N_DEV - 1, N_DEV)
        right = lax.rem(my + 1, N_DEV)

        barrier = pltpu.get_barrier_semaphore()
        for nbr in (left, right):
            pl.semaphore_signal(barrier, inc=1, device_id=(nbr,),
                                device_id_type=pl.DeviceIdType.MESH)
        pl.semaphore_wait(barrier, 2)

        wq_bufs[my] = wq_ref[...]
        wo_bufs[my] = wo_ref[...]

        xv = x_ref[...].reshape(BS, DM)

        qb = lax.broadcasted_iota(jnp.int32, (SQ, SKV), 0) // 64
        kb = lax.broadcasted_iota(jnp.int32, (SQ, SKV), 1) // 64
        mask = (qb == kb)[None, None]

        def compute(j):
            q = lax.dot(xv, wq_bufs[j], preferred_element_type=jnp.float32)
            q4 = q.astype(jnp.bfloat16).reshape(B, SQ, HQ, DH)
            s = jnp.einsum('bqhd,bkhd->bhqk', q4, k_ref[j],
                           preferred_element_type=jnp.float32) * 0.125
            s = jnp.where(mask, s, -1e9)
            w = jnp.exp(s - jnp.max(s, axis=-1, keepdims=True))
            w = w / jnp.sum(w, axis=-1, keepdims=True)
            ctx = jnp.einsum('bhqk,bkhd->bqhd', w.astype(jnp.bfloat16),
                             v_ref[j], preferred_element_type=jnp.float32)
            c2 = ctx.reshape(BS, CHUNK).astype(jnp.bfloat16)
            return lax.dot(c2, wo_bufs[j], preferred_element_type=jnp.float32)

        acc_ref[...] = compute(my)

        pl.semaphore_signal(credit_sem, inc=1)

        def hop(h, carry):
            src = lax.rem(my - h + N_DEV, N_DEV)
            dst = lax.rem(my - h - 1 + N_DEV, N_DEV)
            pl.semaphore_wait(credit_sem, 1)
            r_wq = pltpu.make_async_remote_copy(
                src_ref=wq_bufs.at[src], dst_ref=wq_bufs.at[src],
                send_sem=send_sems.at[0], recv_sem=recv_sems.at[0],
                device_id=(right,), device_id_type=pl.DeviceIdType.MESH)
            r_wo = pltpu.make_async_remote_copy(
                src_ref=wo_bufs.at[src], dst_ref=wo_bufs.at[src],
                send_sem=send_sems.at[1], recv_sem=recv_sems.at[1],
                device_id=(right,), device_id_type=pl.DeviceIdType.MESH)
            r_wq.start()
            r_wo.start()
            r_wq.wait()
            r_wo.wait()
            pl.semaphore_signal(credit_sem, inc=1, device_id=(left,),
                                device_id_type=pl.DeviceIdType.MESH)
            acc_ref[...] += compute(dst)
            return carry

        lax.fori_loop(0, N_DEV - 1, hop, None)
        pl.semaphore_wait(credit_sem, 1)

        out_ref[...] = acc_ref[...].reshape(B, SQ, DM)

    return pl.pallas_call(
        body,
        out_shape=jax.ShapeDtypeStruct((B, SQ, DM), jnp.float32),
        in_specs=[pl.BlockSpec(memory_space=pltpu.VMEM)] * 5,
        out_specs=pl.BlockSpec(memory_space=pltpu.VMEM),
        scratch_shapes=[
            pltpu.VMEM((N_DEV, DM, CHUNK), jnp.bfloat16),
            pltpu.VMEM((N_DEV, CHUNK, DM), jnp.bfloat16),
            pltpu.VMEM((BS, DM), jnp.float32),
            pltpu.SemaphoreType.DMA((2,)),
            pltpu.SemaphoreType.DMA((2,)),
            pltpu.SemaphoreType.REGULAR,
        ],
        compiler_params=pltpu.CompilerParams(collective_id=0),
    )(xb, wqb, k5, v5, wob)


# baseline (device time: 180762 ns/iter reference)
import jax
import jax.numpy as jnp
from jax import lax
from jax.experimental import pallas as pl
from jax.experimental.pallas import tpu as pltpu

N_DEV = 16
B = 2
SQ = 256
SKV = 256
HQ = 4
DH = 64
CHUNK = HQ * DH
DM = 512
BS = B * SQ


def kernel(x, Wq, K_ext, V_ext, Wo):
    xb = x.astype(jnp.bfloat16)
    wqb = Wq.astype(jnp.bfloat16)
    wob = Wo.astype(jnp.bfloat16)
    k5 = K_ext.astype(jnp.bfloat16).reshape(B, SKV, N_DEV, HQ, DH)
    k5 = k5.transpose(2, 0, 3, 4, 1)
    v5 = V_ext.astype(jnp.bfloat16).reshape(B, SKV, N_DEV, HQ, DH)
    v5 = v5.transpose(2, 0, 3, 1, 4)

    def body(x_ref, wq_ref, k_ref, v_ref, wo_ref, out_ref,
             wq_bufs, wo_bufs, acc_ref, send_sems, recv_sems, credit_sem):
        my = lax.axis_index("i")
        left = lax.rem(my + N_DEV - 1, N_DEV)
        right = lax.rem(my + 1, N_DEV)

        barrier = pltpu.get_barrier_semaphore()
        for nbr in (left, right):
            pl.semaphore_signal(barrier, inc=1, device_id=(nbr,),
                                device_id_type=pl.DeviceIdType.MESH)
        pl.semaphore_wait(barrier, 2)

        wq_bufs[my] = wq_ref[...]
        wo_bufs[my] = wo_ref[...]

        xv = x_ref[...].reshape(BS, DM)

        qb = lax.broadcasted_iota(jnp.int32, (SQ, SKV), 0) // 64
        kb = lax.broadcasted_iota(jnp.int32, (SQ, SKV), 1) // 64
        mask = qb == kb

        def compute(j):
            q = lax.dot(xv, wq_bufs[j], preferred_element_type=jnp.float32)
            qb16 = q.astype(jnp.bfloat16)
            ctx_rows = []
            for b in range(B):
                ctx_h = []
                for h in range(HQ):
                    qbh = qb16[b * SQ:(b + 1) * SQ, h * DH:(h + 1) * DH]
                    s = lax.dot(qbh, k_ref[j, b, h],
                                preferred_element_type=jnp.float32) * 0.125
                    s = jnp.where(mask, s, -1e9)
                    w = jnp.exp(s - jnp.max(s, axis=-1, keepdims=True))
                    w = w / jnp.sum(w, axis=-1, keepdims=True)
                    ctx_h.append(lax.dot(w.astype(jnp.bfloat16), v_ref[j, b, h],
                                         preferred_element_type=jnp.float32))
                ctx_rows.append(jnp.concatenate(ctx_h, axis=1))
            c2 = jnp.concatenate(ctx_rows, axis=0).astype(jnp.bfloat16)
            return lax.dot(c2, wo_bufs[j], preferred_element_type=jnp.float32)

        acc_ref[...] = compute(my)

        pl.semaphore_signal(credit_sem, inc=1)

        def hop(h, carry):
            src = lax.rem(my - h + N_DEV, N_DEV)
            dst = lax.rem(my - h - 1 + N_DEV, N_DEV)
            pl.semaphore_wait(credit_sem, 1)
            r_wq = pltpu.make_async_remote_copy(
                src_ref=wq_bufs.at[src], dst_ref=wq_bufs.at[src],
                send_sem=send_sems.at[0], recv_sem=recv_sems.at[0],
                device_id=(right,), device_id_type=pl.DeviceIdType.MESH)
            r_wo = pltpu.make_async_remote_copy(
                src_ref=wo_bufs.at[src], dst_ref=wo_bufs.at[src],
                send_sem=send_sems.at[1], recv_sem=recv_sems.at[1],
                device_id=(right,), device_id_type=pl.DeviceIdType.MESH)
            r_wq.start()
            r_wo.start()
            r_wq.wait()
            r_wo.wait()
            pl.semaphore_signal(credit_sem, inc=1, device_id=(left,),
                                device_id_type=pl.DeviceIdType.MESH)
            acc_ref[...] += compute(dst)
            return carry

        lax.fori_loop(0, N_DEV - 1, hop, None)
        pl.semaphore_wait(credit_sem, 1)

        out_ref[...] = acc_ref[...].reshape(B, SQ, DM)

    return pl.pallas_call(
        body,
        out_shape=jax.ShapeDtypeStruct((B, SQ, DM), jnp.float32),
        in_specs=[pl.BlockSpec(memory_space=pltpu.VMEM)] * 5,
        out_specs=pl.BlockSpec(memory_space=pltpu.VMEM),
        scratch_shapes=[
            pltpu.VMEM((N_DEV, DM, CHUNK), jnp.bfloat16),
            pltpu.VMEM((N_DEV, CHUNK, DM), jnp.bfloat16),
            pltpu.VMEM((BS, DM), jnp.float32),
            pltpu.SemaphoreType.DMA((2,)),
            pltpu.SemaphoreType.DMA((2,)),
            pltpu.SemaphoreType.REGULAR,
        ],
        compiler_params=pltpu.CompilerParams(collective_id=0),
    )(xb, wqb, k5, v5, wob)


# device time: 102904 ns/iter; 1.7566x vs baseline; 1.7566x over previous
import jax
import jax.numpy as jnp
from jax import lax
from jax.experimental import pallas as pl
from jax.experimental.pallas import tpu as pltpu

N_DEV = 16
B = 2
SQ = 256
SKV = 256
HQ = 4
DH = 64
CHUNK = HQ * DH
DM = 512
BS = B * SQ


def kernel(x, Wq, K_ext, V_ext, Wo):
    xb = x.astype(jnp.bfloat16)
    wqb = Wq.astype(jnp.bfloat16)
    wob = Wo.astype(jnp.bfloat16)
    k5 = K_ext.astype(jnp.bfloat16).reshape(B, SKV, N_DEV, HQ, DH)
    k5 = k5.transpose(2, 0, 3, 4, 1)
    v5 = V_ext.astype(jnp.bfloat16).reshape(B, SKV, N_DEV, HQ, DH)
    v5 = v5.transpose(2, 0, 3, 1, 4)

    def body(x_ref, wq_ref, k_ref, v_ref, wo_ref, out_ref,
             wq_bufs, wo_bufs, acc_ref, send_sems, recv_sems,
             credit_r, credit_l):
        my = lax.axis_index("i")
        left = lax.rem(my + N_DEV - 1, N_DEV)
        right = lax.rem(my + 1, N_DEV)

        barrier = pltpu.get_barrier_semaphore()
        for nbr in (left, right):
            pl.semaphore_signal(barrier, inc=1, device_id=(nbr,),
                                device_id_type=pl.DeviceIdType.MESH)
        pl.semaphore_wait(barrier, 2)

        wq_bufs[my] = wq_ref[...]
        wo_bufs[my] = wo_ref[...]

        xv = x_ref[...].reshape(BS, DM)

        qb = lax.broadcasted_iota(jnp.int32, (SQ, SKV), 0) // 64
        kb = lax.broadcasted_iota(jnp.int32, (SQ, SKV), 1) // 64
        mask = qb == kb

        def compute(j):
            q = lax.dot(xv, wq_bufs[j], preferred_element_type=jnp.float32)
            qb16 = q.astype(jnp.bfloat16)
            ctx_rows = []
            for b in range(B):
                ctx_h = []
                for h in range(HQ):
                    qbh = qb16[b * SQ:(b + 1) * SQ, h * DH:(h + 1) * DH]
                    s = lax.dot(qbh, k_ref[j, b, h],
                                preferred_element_type=jnp.float32) * 0.125
                    s = jnp.where(mask, s, -1e9)
                    w = jnp.exp(s - jnp.max(s, axis=-1, keepdims=True))
                    w = w / jnp.sum(w, axis=-1, keepdims=True)
                    ctx_h.append(lax.dot(w.astype(jnp.bfloat16), v_ref[j, b, h],
                                         preferred_element_type=jnp.float32))
                ctx_rows.append(jnp.concatenate(ctx_h, axis=1))
            c2 = jnp.concatenate(ctx_rows, axis=0).astype(jnp.bfloat16)
            return lax.dot(c2, wo_bufs[j], preferred_element_type=jnp.float32)

        acc_ref[...] = jnp.zeros((BS, DM), jnp.float32)

        pl.semaphore_signal(credit_r, inc=1)
        pl.semaphore_signal(credit_l, inc=1)

        def rdma(bufs, slot, s, r, tgt):
            return pltpu.make_async_remote_copy(
                src_ref=bufs.at[slot], dst_ref=bufs.at[slot],
                send_sem=send_sems.at[s], recv_sem=recv_sems.at[r],
                device_id=(tgt,), device_id_type=pl.DeviceIdType.MESH)

        HOPS_R = N_DEV // 2
        HOPS_L = N_DEV // 2 - 1

        def hop(h, carry):
            rsrc = lax.rem(my - h + N_DEV, N_DEV)
            lsrc = lax.rem(my + h, N_DEV)
            pl.semaphore_wait(credit_r, 1)
            r_wq = rdma(wq_bufs, rsrc, 0, 0, right)
            r_wo = rdma(wo_bufs, rsrc, 1, 1, right)
            r_wq.start()
            r_wo.start()
            l_wq = rdma(wq_bufs, lsrc, 2, 2, left)
            l_wo = rdma(wo_bufs, lsrc, 3, 3, left)

            @pl.when(h < HOPS_L)
            def _():
                pl.semaphore_wait(credit_l, 1)
                l_wq.start()
                l_wo.start()

            acc_ref[...] += compute(rsrc)

            @pl.when(h > 0)
            def _():
                acc_ref[...] += compute(lsrc)

            r_wq.wait()
            r_wo.wait()
            pl.semaphore_signal(credit_r, inc=1, device_id=(left,),
                                device_id_type=pl.DeviceIdType.MESH)

            @pl.when(h < HOPS_L)
            def _():
                l_wq.wait()
                l_wo.wait()
                pl.semaphore_signal(credit_l, inc=1, device_id=(right,),
                                    device_id_type=pl.DeviceIdType.MESH)

            return carry

        lax.fori_loop(0, HOPS_R, hop, None)
        acc_ref[...] += compute(lax.rem(my - HOPS_R + N_DEV, N_DEV))
        pl.semaphore_wait(credit_r, 1)
        pl.semaphore_wait(credit_l, 1)

        out_ref[...] = acc_ref[...].reshape(B, SQ, DM)

    return pl.pallas_call(
        body,
        out_shape=jax.ShapeDtypeStruct((B, SQ, DM), jnp.float32),
        in_specs=[pl.BlockSpec(memory_space=pltpu.VMEM)] * 5,
        out_specs=pl.BlockSpec(memory_space=pltpu.VMEM),
        scratch_shapes=[
            pltpu.VMEM((N_DEV, DM, CHUNK), jnp.bfloat16),
            pltpu.VMEM((N_DEV, CHUNK, DM), jnp.bfloat16),
            pltpu.VMEM((BS, DM), jnp.float32),
            pltpu.SemaphoreType.DMA((4,)),
            pltpu.SemaphoreType.DMA((4,)),
            pltpu.SemaphoreType.REGULAR,
            pltpu.SemaphoreType.REGULAR,
        ],
        compiler_params=pltpu.CompilerParams(collective_id=0),
    )(xb, wqb, k5, v5, wob)
